# baseline (speedup 1.0000x reference)
import functools

import jax
import jax.numpy as jnp
from jax import lax
from jax.experimental import pallas as pl
from jax.experimental.pallas import tpu as pltpu
from jax.experimental.pallas import tpu_sc as plsc
from jax.experimental import layout as jex_layout

N = 10000
E = 320000
D = 128
DH = D // 2
G = 64

NC = 2
NS = 16
NW = NC * NS
CHUNK = 128
EROWS = 2560
EPAD = EROWS * CHUNK
RPW = EROWS // NW
NPAD = 10240
NPT = NPAD // NS
NBUF = 4
LOOK = 2

_F32 = jnp.float32
_I32 = jnp.int32


def _sc_mesh():
    return plsc.VectorSubcoreMesh(core_axis_name="c", subcore_axis_name="s")


@functools.partial(
    pl.kernel,
    out_type=jax.ShapeDtypeStruct((NC * NPAD,), _F32),
    mesh=_sc_mesh(),
    compiler_params=pltpu.CompilerParams(needs_layout_passes=False),
    scratch_types=[
        pltpu.VMEM((RPW, CHUNK), _I32),
        pltpu.VMEM((RPW, CHUNK), _F32),
        pltpu.VMEM((NPT,), _F32),
        pltpu.VMEM_SHARED((NPAD,), _F32),
    ],
)
def _deg_kernel(dst_hbm, ew_hbm, out_hbm, dst_v, ew_v, buf_v, deg_s):
    c = lax.axis_index("c")
    s = lax.axis_index("s")
    w = s * NC + c
    zero16 = jnp.zeros((16,), _F32)

    @pl.loop(0, NPT // 16)
    def _(i):
        buf_v[pl.ds(i * 16, 16)] = zero16

    pltpu.sync_copy(buf_v, deg_s.at[pl.ds(s * NPT, NPT)])
    plsc.subcore_barrier()
    pltpu.sync_copy(dst_hbm.at[pl.ds(w * RPW, RPW)], dst_v)
    pltpu.sync_copy(ew_hbm.at[pl.ds(w * RPW, RPW)], ew_v)

    @pl.loop(0, RPW)
    def _(j):
        pltpu.sync_copy(ew_v.at[j], deg_s.at[dst_v.at[j]], add=True)

    plsc.subcore_barrier()
    pltpu.sync_copy(deg_s.at[pl.ds(s * NPT, NPT)], buf_v)
    pltpu.sync_copy(buf_v, out_hbm.at[pl.ds(c * NPAD + s * NPT, NPT)])


@functools.partial(
    pl.kernel,
    out_type=jax.ShapeDtypeStruct((NC * NPAD, DH), _F32),
    mesh=_sc_mesh(),
    compiler_params=pltpu.CompilerParams(needs_layout_passes=False,
                                         use_tc_tiling_on_sc=False),
    scratch_types=[
        pltpu.VMEM((RPW, CHUNK), _I32),
        pltpu.VMEM((RPW, CHUNK), _I32),
        pltpu.VMEM((RPW, CHUNK), _F32),
        pltpu.VMEM((CHUNK, DH), _F32),
        pltpu.VMEM((CHUNK, DH), _F32),
        pltpu.VMEM((CHUNK, DH), _F32),
        pltpu.VMEM((CHUNK, DH), _F32),
        pltpu.VMEM_SHARED((NPAD, DH), _F32),
        pltpu.SemaphoreType.DMA,
        pltpu.SemaphoreType.DMA,
        pltpu.SemaphoreType.DMA,
        pltpu.SemaphoreType.DMA,
        pltpu.SemaphoreType.DMA,
        pltpu.SemaphoreType.DMA,
        pltpu.SemaphoreType.DMA,
        pltpu.SemaphoreType.DMA,
    ],
)
def _spmm_kernel(h128_hbm, src_hbm, dst_hbm, ce_hbm, out128_hbm,
                 src_v, dst_v, c_v, r0, r1, r2, r3,
                 acc_s, g0, g1, g2, g3, s0, s1, s2, s3):
    c = lax.axis_index("c")
    s = lax.axis_index("s")
    w = s * NC + c
    h_hbm = h128_hbm
    out_hbm = out128_hbm
    zero16 = jnp.zeros((16,), _F32)
    rows = (r0, r1, r2, r3)
    gsem = (g0, g1, g2, g3)
    ssem = (s0, s1, s2, s3)

    def issue_gather(jj, b):
        pltpu.async_copy(h_hbm.at[src_v.at[jj]], rows[b], gsem[b])

    def wait_gather(b):
        pltpu.make_async_copy(h_hbm.at[src_v.at[0]], rows[b], gsem[b]).wait()

    def issue_scatter(jj, b):
        pltpu.async_copy(rows[b], acc_s.at[dst_v.at[jj]], ssem[b], add=True)

    def wait_scatter(b):
        pltpu.make_async_copy(rows[b], acc_s.at[dst_v.at[0]], ssem[b]).wait()

    @pl.loop(0, CHUNK)
    def _(r):
        for k in range(DH // 16):
            r0[r, pl.ds(k * 16, 16)] = zero16

    for q in range(NPT // CHUNK):
        pltpu.sync_copy(r0, acc_s.at[pl.ds(s * NPT + q * CHUNK, CHUNK)])
    plsc.subcore_barrier()

    pltpu.sync_copy(src_hbm.at[pl.ds(w * RPW, RPW)], src_v)
    pltpu.sync_copy(dst_hbm.at[pl.ds(w * RPW, RPW)], dst_v)
    pltpu.sync_copy(ce_hbm.at[pl.ds(w * RPW, RPW)], c_v)

    off = c * N

    @pl.loop(0, RPW)
    def _(j):
        for g in range(CHUNK // 16):
            sl = pl.ds(g * 16, 16)
            src_v[j, sl] = src_v[j, sl] + off

    for b in range(LOOK):
        issue_gather(b, b)

    @pl.loop(0, RPW // NBUF)
    def _(p):
        j0 = p * NBUF
        for b in range(NBUF):
            j = j0 + b
            wait_gather(b)
            rb = rows[b]

            @pl.loop(0, CHUNK // 16)
            def _(gi):
                cv = c_v[j, pl.ds(gi * 16, 16)]
                bs = gi * 16
                for l in range(16):
                    cs = cv[l]
                    for k in range(DH // 16):
                        sl = pl.ds(k * 16, 16)
                        rb[bs + l, sl] = rb[bs + l, sl] * cs

            issue_scatter(j, b)
            f = j + LOOK
            bf = (b + LOOK) % NBUF

            @pl.when(f < RPW)
            def _():
                @pl.when(f >= NBUF)
                def _():
                    wait_scatter(bf)
                issue_gather(f, bf)

    for b in range(NBUF):
        wait_scatter(b)
    plsc.subcore_barrier()
    for q in range(NPT // CHUNK):
        pltpu.sync_copy(acc_s.at[pl.ds(s * NPT + q * CHUNK, CHUNK)], r0)
        pltpu.sync_copy(
            r0, out_hbm.at[pl.ds(c * NPAD + s * NPT + q * CHUNK, CHUNK)])


def _tc1_body(x_ref, w_ref, degcol_ref, h_ref):
    dis = lax.rsqrt(degcol_ref[...])
    h_ref[...] = dis * lax.dot(x_ref[...], w_ref[...],
                               precision=lax.Precision.HIGHEST)


_tc1 = pl.pallas_call(
    _tc1_body,
    out_shape=jax.ShapeDtypeStruct((N, D), _F32),
)


def _tc2_body(aL_ref, aR_ref, h_ref, degcol_ref, b_ref, w_ref, out_ref):
    dis = lax.rsqrt(degcol_ref[...])
    acc = lax.concatenate([aL_ref[...], aR_ref[...]], dimension=1)
    z = dis * (acc + h_ref[...]) + b_ref[...]
    out_ref[...] = dis * lax.dot(jnp.maximum(z, 0.0), w_ref[...],
                                 precision=lax.Precision.HIGHEST)


_tc2 = pl.pallas_call(
    _tc2_body,
    out_shape=jax.ShapeDtypeStruct((N, D), _F32),
)


def _tc3_body(aL_ref, aR_ref, h_ref, degcol_ref, b_ref, batch_ref, out_ref):
    dis = lax.rsqrt(degcol_ref[...])
    acc = lax.concatenate([aL_ref[...], aR_ref[...]], dimension=1)
    z = dis * (acc + h_ref[...]) + b_ref[...]
    hr = jnp.maximum(z, 0.0)
    oh = (batch_ref[...] ==
          lax.broadcasted_iota(_I32, (N, G), 1)).astype(_F32)
    sums = lax.dot_general(oh, hr, (((0,), (0,)), ((), ())),
                           precision=lax.Precision.HIGHEST)
    counts = lax.dot_general(oh, jnp.ones((N, 1), _F32),
                             (((0,), (0,)), ((), ())),
                             precision=lax.Precision.HIGHEST)
    out_ref[...] = sums / jnp.maximum(counts, 1.0)


_tc3 = pl.pallas_call(
    _tc3_body,
    out_shape=jax.ShapeDtypeStruct((G, D), _F32),
)


@jax.jit
def kernel(node_features, edge_index, edge_weight, batch, W1, b1, W2, b2):
    x = node_features.astype(_F32)
    src = edge_index[0].astype(_I32)
    dst = edge_index[1].astype(_I32)
    ew = edge_weight.astype(_F32)

    pad = EPAD - E
    fill = jnp.arange(pad, dtype=_I32) % N
    srcp = jnp.concatenate([src, fill]).reshape(EROWS, CHUNK)
    dstp = jnp.concatenate([dst, fill]).reshape(EROWS, CHUNK)
    ewp = jnp.concatenate([ew, jnp.zeros((pad,), _F32)]).reshape(EROWS, CHUNK)

    degp = _deg_kernel(dstp, ewp)
    degcol = (degp[:NPAD] + degp[NPAD:] + 1.0).reshape(NPAD, 1)[:N]

    _lin = jex_layout.Layout(major_to_minor=(0, 1), tiling=((8,),))

    def split_tab(h):
        t = jnp.concatenate([h[:, :DH], h[:, DH:]], axis=0)
        return jex_layout.with_layout_constraint(t, _lin)

    h1 = _tc1(x, W1, degcol)
    p1 = _spmm_kernel(split_tab(h1), srcp, dstp, ewp)
    p1 = jex_layout.with_layout_constraint(p1, _lin)
    h2 = _tc2(p1[:N], p1[NPAD:NPAD + N], h1, degcol, b1.reshape(1, D), W2)
    p2 = _spmm_kernel(split_tab(h2), srcp, dstp, ewp)
    p2 = jex_layout.with_layout_constraint(p2, _lin)
    out = _tc3(p2[:N], p2[NPAD:NPAD + N], h2, degcol, b2.reshape(1, D),
               batch.astype(_I32).reshape(N, 1))
    return out


# speedup vs baseline: 1.1193x; 1.1193x over previous
import functools

import jax
import jax.numpy as jnp
from jax import lax
from jax.experimental import pallas as pl
from jax.experimental.pallas import tpu as pltpu
from jax.experimental.pallas import tpu_sc as plsc
from jax.experimental import layout as jex_layout

N = 10000
E = 320000
D = 128
DH = D // 2
G = 64

NC = 2
NS = 16
NW = NC * NS
CHUNK = 128
EROWS = 2560
EPAD = EROWS * CHUNK
RPW = EROWS // NW
NPAD = 10240
NPT = NPAD // NS
NBUF = 5
LOOK = 3

_F32 = jnp.float32
_I32 = jnp.int32


def _sc_mesh():
    return plsc.VectorSubcoreMesh(core_axis_name="c", subcore_axis_name="s")


@functools.partial(
    pl.kernel,
    out_type=jax.ShapeDtypeStruct((NC * NPAD,), _F32),
    mesh=_sc_mesh(),
    compiler_params=pltpu.CompilerParams(needs_layout_passes=False),
    scratch_types=[
        pltpu.VMEM((RPW, CHUNK), _I32),
        pltpu.VMEM((RPW, CHUNK), _F32),
        pltpu.VMEM((NPT,), _F32),
        pltpu.VMEM_SHARED((NPAD,), _F32),
    ],
)
def _deg_kernel(dst_hbm, ew_hbm, out_hbm, dst_v, ew_v, buf_v, deg_s):
    c = lax.axis_index("c")
    s = lax.axis_index("s")
    w = s * NC + c
    zero16 = jnp.zeros((16,), _F32)

    @pl.loop(0, NPT // 16)
    def _(i):
        buf_v[pl.ds(i * 16, 16)] = zero16

    pltpu.sync_copy(buf_v, deg_s.at[pl.ds(s * NPT, NPT)])
    plsc.subcore_barrier()
    pltpu.sync_copy(dst_hbm.at[pl.ds(w * RPW, RPW)], dst_v)
    pltpu.sync_copy(ew_hbm.at[pl.ds(w * RPW, RPW)], ew_v)

    @pl.loop(0, RPW)
    def _(j):
        pltpu.sync_copy(ew_v.at[j], deg_s.at[dst_v.at[j]], add=True)

    plsc.subcore_barrier()
    pltpu.sync_copy(deg_s.at[pl.ds(s * NPT, NPT)], buf_v)
    pltpu.sync_copy(buf_v, out_hbm.at[pl.ds(c * NPAD + s * NPT, NPT)])


@functools.partial(
    pl.kernel,
    out_type=jax.ShapeDtypeStruct((EROWS, CHUNK), _F32),
    mesh=_sc_mesh(),
    compiler_params=pltpu.CompilerParams(needs_layout_passes=False),
    scratch_types=[
        pltpu.VMEM((NPAD,), _F32),
        pltpu.VMEM((RPW, CHUNK), _I32),
        pltpu.VMEM((RPW, CHUNK), _I32),
        pltpu.VMEM((RPW, CHUNK), _F32),
        pltpu.VMEM((RPW, CHUNK), _F32),
    ],
)
def _ce_kernel(src_hbm, dst_hbm, ew_hbm, dis_hbm, out_hbm,
               dis_v, src_v, dst_v, ew_v, c_v):
    c = lax.axis_index("c")
    s = lax.axis_index("s")
    w = s * NC + c
    pltpu.sync_copy(dis_hbm, dis_v)
    pltpu.sync_copy(src_hbm.at[pl.ds(w * RPW, RPW)], src_v)
    pltpu.sync_copy(dst_hbm.at[pl.ds(w * RPW, RPW)], dst_v)
    pltpu.sync_copy(ew_hbm.at[pl.ds(w * RPW, RPW)], ew_v)

    @pl.loop(0, RPW)
    def _(j):
        for g in range(CHUNK // 16):
            sl = pl.ds(g * 16, 16)
            a = plsc.load_gather(dis_v, [src_v[j, sl]])
            b = plsc.load_gather(dis_v, [dst_v[j, sl]])
            c_v[j, sl] = ew_v[j, sl] * a * b

    pltpu.sync_copy(c_v, out_hbm.at[pl.ds(w * RPW, RPW)])


@functools.partial(
    pl.kernel,
    out_type=jax.ShapeDtypeStruct((NC * NPAD, DH), _F32),
    mesh=_sc_mesh(),
    compiler_params=pltpu.CompilerParams(needs_layout_passes=False,
                                         use_tc_tiling_on_sc=False),
    scratch_types=[
        pltpu.VMEM((RPW, CHUNK), _I32),
        pltpu.VMEM((RPW, CHUNK), _I32),
        pltpu.VMEM((RPW, CHUNK), _F32),
        pltpu.VMEM((CHUNK, DH), _F32),
        pltpu.VMEM((CHUNK, DH), _F32),
        pltpu.VMEM((CHUNK, DH), _F32),
        pltpu.VMEM((CHUNK, DH), _F32),
        pltpu.VMEM((CHUNK, DH), _F32),
        pltpu.VMEM_SHARED((NPAD, DH), _F32),
        pltpu.SemaphoreType.DMA,
        pltpu.SemaphoreType.DMA,
        pltpu.SemaphoreType.DMA,
        pltpu.SemaphoreType.DMA,
        pltpu.SemaphoreType.DMA,
        pltpu.SemaphoreType.DMA,
        pltpu.SemaphoreType.DMA,
        pltpu.SemaphoreType.DMA,
        pltpu.SemaphoreType.DMA,
        pltpu.SemaphoreType.DMA,
    ],
)
def _spmm_kernel(h128_hbm, src_hbm, dst_hbm, ce_hbm, out128_hbm,
                 src_v, dst_v, c_v, r0, r1, r2, r3, r4,
                 acc_s, g0, g1, g2, g3, g4, s0, s1, s2, s3, s4):
    c = lax.axis_index("c")
    s = lax.axis_index("s")
    w = s * NC + c
    h_hbm = h128_hbm
    out_hbm = out128_hbm
    zero16 = jnp.zeros((16,), _F32)
    rows = (r0, r1, r2, r3, r4)
    gsem = (g0, g1, g2, g3, g4)
    ssem = (s0, s1, s2, s3, s4)

    def issue_gather(jj, b):
        pltpu.async_copy(h_hbm.at[src_v.at[jj]], rows[b], gsem[b])

    def wait_gather(b):
        pltpu.make_async_copy(h_hbm.at[src_v.at[0]], rows[b], gsem[b]).wait()

    def issue_scatter(jj, b):
        pltpu.async_copy(rows[b], acc_s.at[dst_v.at[jj]], ssem[b], add=True)

    def wait_scatter(b):
        pltpu.make_async_copy(rows[b], acc_s.at[dst_v.at[0]], ssem[b]).wait()

    @pl.loop(0, CHUNK)
    def _(r):
        for k in range(DH // 16):
            r0[r, pl.ds(k * 16, 16)] = zero16

    for q in range(NPT // CHUNK):
        pltpu.sync_copy(r0, acc_s.at[pl.ds(s * NPT + q * CHUNK, CHUNK)])
    plsc.subcore_barrier()

    pltpu.sync_copy(src_hbm.at[pl.ds(w * RPW, RPW)], src_v)
    pltpu.sync_copy(dst_hbm.at[pl.ds(w * RPW, RPW)], dst_v)
    pltpu.sync_copy(ce_hbm.at[pl.ds(w * RPW, RPW)], c_v)

    off = c * N

    @pl.loop(0, RPW)
    def _(j):
        for g in range(CHUNK // 16):
            sl = pl.ds(g * 16, 16)
            src_v[j, sl] = src_v[j, sl] + off

    for b in range(LOOK):
        issue_gather(b, b)

    @pl.loop(0, RPW // NBUF)
    def _(p):
        j0 = p * NBUF
        for b in range(NBUF):
            j = j0 + b
            wait_gather(b)
            rb = rows[b]

            @pl.loop(0, CHUNK // 16)
            def _(gi):
                cv = c_v[j, pl.ds(gi * 16, 16)]
                bs = gi * 16
                for l in range(16):
                    cs = cv[l]
                    for k in range(DH // 16):
                        sl = pl.ds(k * 16, 16)
                        rb[bs + l, sl] = rb[bs + l, sl] * cs

            issue_scatter(j, b)
            f = j + LOOK
            bf = (b + LOOK) % NBUF

            @pl.when(f < RPW)
            def _():
                @pl.when(f >= NBUF)
                def _():
                    wait_scatter(bf)
                issue_gather(f, bf)

    for b in range(NBUF):
        wait_scatter(b)
    plsc.subcore_barrier()
    for q in range(NPT // CHUNK):
        pltpu.sync_copy(acc_s.at[pl.ds(s * NPT + q * CHUNK, CHUNK)], r0)
        pltpu.sync_copy(
            r0, out_hbm.at[pl.ds(c * NPAD + s * NPT + q * CHUNK, CHUNK)])


def _tc1_body(x_ref, w_ref, degp_ref, h_ref, dis_ref, dis2_ref):
    deg = degp_ref[0:1, :] + degp_ref[1:2, :] + 1.0
    dis_ref[...] = lax.rsqrt(deg)
    dis2_ref[...] = 1.0 / deg
    h_ref[...] = lax.dot(x_ref[...], w_ref[...],
                         precision=lax.Precision.HIGHEST)


_tc1 = pl.pallas_call(
    _tc1_body,
    out_shape=(
        jax.ShapeDtypeStruct((N, D), _F32),
        jax.ShapeDtypeStruct((1, NPAD), _F32),
        jax.ShapeDtypeStruct((1, NPAD), _F32),
    ),
)


def _tc2_body(aL_ref, aR_ref, h_ref, dis2_ref, b_ref, w_ref, out_ref):
    acc = lax.concatenate([aL_ref[...], aR_ref[...]], dimension=1)
    z = acc + dis2_ref[...] * h_ref[...] + b_ref[...]
    out_ref[...] = lax.dot(jnp.maximum(z, 0.0), w_ref[...],
                           precision=lax.Precision.HIGHEST)


_tc2 = pl.pallas_call(
    _tc2_body,
    out_shape=jax.ShapeDtypeStruct((N, D), _F32),
)


def _tc3_body(aL_ref, aR_ref, h_ref, dis2_ref, b_ref, batch_ref, out_ref):
    acc = lax.concatenate([aL_ref[...], aR_ref[...]], dimension=1)
    z = acc + dis2_ref[...] * h_ref[...] + b_ref[...]
    hr = jnp.maximum(z, 0.0)
    oh = (batch_ref[...] ==
          lax.broadcasted_iota(_I32, (N, G), 1)).astype(_F32)
    sums = lax.dot_general(oh, hr, (((0,), (0,)), ((), ())),
                           precision=lax.Precision.HIGHEST)
    counts = lax.dot_general(oh, jnp.ones((N, 1), _F32),
                             (((0,), (0,)), ((), ())),
                             precision=lax.Precision.HIGHEST)
    out_ref[...] = sums / jnp.maximum(counts, 1.0)


_tc3 = pl.pallas_call(
    _tc3_body,
    out_shape=jax.ShapeDtypeStruct((G, D), _F32),
)


@jax.jit
def kernel(node_features, edge_index, edge_weight, batch, W1, b1, W2, b2):
    x = node_features.astype(_F32)
    src = edge_index[0].astype(_I32)
    dst = edge_index[1].astype(_I32)
    ew = edge_weight.astype(_F32)

    pad = EPAD - E
    fill = jnp.arange(pad, dtype=_I32) % N
    srcp = jnp.concatenate([src, fill]).reshape(EROWS, CHUNK)
    dstp = jnp.concatenate([dst, fill]).reshape(EROWS, CHUNK)
    ewp = jnp.concatenate([ew, jnp.zeros((pad,), _F32)]).reshape(EROWS, CHUNK)

    degp = _deg_kernel(dstp, ewp).reshape(NC, NPAD)
    h1, dis, dis2 = _tc1(x, W1, degp)
    ce = _ce_kernel(srcp, dstp, ewp, dis.reshape(NPAD))
    dis2c = dis2.reshape(NPAD, 1)[:N]

    _lin = jex_layout.Layout(major_to_minor=(0, 1), tiling=((8,),))

    def split_tab(h):
        t = jnp.concatenate([h[:, :DH], h[:, DH:]], axis=0)
        return jex_layout.with_layout_constraint(t, _lin)

    p1 = _spmm_kernel(split_tab(h1), srcp, dstp, ce)
    p1 = jex_layout.with_layout_constraint(p1, _lin)
    h2 = _tc2(p1[:N], p1[NPAD:NPAD + N], h1, dis2c, b1.reshape(1, D), W2)
    p2 = _spmm_kernel(split_tab(h2), srcp, dstp, ce)
    p2 = jex_layout.with_layout_constraint(p2, _lin)
    out = _tc3(p2[:N], p2[NPAD:NPAD + N], h2, dis2c, b2.reshape(1, D),
               batch.astype(_I32).reshape(N, 1))
    return out
